# baseline (speedup 1.0000x reference)
import jax
import jax.numpy as jnp
from jax.experimental import pallas as pl
from jax.experimental.pallas import tpu as pltpu

_B, _C, _K, _H, _W = 64, 512, 64, 64, 64
_HW = _H * _W
_R = _HW // _C
_EPS = 1e-12


def _netvlad_kernel(x4_ref, w_ref, b_ref, cent_ref, out_ref, xn_scr):
    x4 = x4_ref[0]
    ssq3 = jnp.sum(x4 * x4, axis=0, keepdims=True)
    rnorm3 = 1.0 / jnp.maximum(jnp.sqrt(ssq3), _EPS)
    xn_bf4 = (x4 * rnorm3).astype(jnp.bfloat16)
    xn2 = xn_bf4.reshape(_C, _HW)

    logits = jax.lax.dot_general(
        w_ref[...].astype(jnp.bfloat16), xn2, (((1,), (0,)), ((), ())),
        preferred_element_type=jnp.float32) + b_ref[...]
    m = jnp.max(logits, axis=0, keepdims=True)
    e = jnp.exp(logits - m)
    a = e / jnp.sum(e, axis=0, keepdims=True)

    xn_scr[...] = xn2
    xfn = xn_scr[...].reshape(_C, _R, _C).reshape(_HW, _C)

    vlad = jax.lax.dot_general(
        a.astype(jnp.bfloat16), xfn, (((1,), (0,)), ((), ())),
        preferred_element_type=jnp.float32)
    vlad = vlad - jnp.sum(a, axis=1, keepdims=True) * cent_ref[...]
    n1 = jnp.sqrt(jnp.sum(vlad * vlad, axis=1, keepdims=True))
    vlad = vlad / jnp.maximum(n1, _EPS)
    n2 = jnp.sqrt(jnp.sum(vlad * vlad))
    out_ref[0] = vlad / jnp.maximum(n2, _EPS)


def kernel(x, conv_w, conv_b, centroids):
    out = pl.pallas_call(
        _netvlad_kernel,
        grid=(_B,),
        in_specs=[
            pl.BlockSpec((1, _C, _H, _W), lambda i: (i, 0, 0, 0)),
            pl.BlockSpec((_K, _C), lambda i: (0, 0)),
            pl.BlockSpec((_K, 1), lambda i: (0, 0)),
            pl.BlockSpec((_K, _C), lambda i: (0, 0)),
        ],
        out_specs=pl.BlockSpec((1, _K, _C), lambda i: (i, 0, 0)),
        out_shape=jax.ShapeDtypeStruct((_B, _K, _C), jnp.float32),
        scratch_shapes=[pltpu.VMEM((_C, _HW), jnp.bfloat16)],
        compiler_params=pltpu.CompilerParams(
            dimension_semantics=("parallel",),
            vmem_limit_bytes=56 * 1024 * 1024,
        ),
        name="netvlad_fused",
    )(x, conv_w, conv_b.reshape(_K, 1), centroids)
    return out.reshape(_B, _K * _C)


# speedup vs baseline: 1.9907x; 1.9907x over previous
import jax
import jax.numpy as jnp
from jax.experimental import pallas as pl
from jax.experimental.pallas import tpu as pltpu

_B, _C, _K, _H, _W = 64, 512, 64, 64, 64
_HW = _H * _W
_R = _HW // _C
_EPS = 1e-12


def _netvlad_kernel(x2_ref, w_ref, b_ref, cent_ref, out_ref):
    x2 = x2_ref[0]
    xb = x2.astype(jnp.bfloat16)
    u = jax.lax.dot_general(
        w_ref[...].astype(jnp.bfloat16), xb, (((1,), (0,)), ((), ())),
        preferred_element_type=jnp.float32)
    ssq = jnp.sum(x2 * x2, axis=0, keepdims=True)
    rnorm = 1.0 / jnp.maximum(jnp.sqrt(ssq), _EPS)
    logits = u * rnorm + b_ref[...]
    m = jnp.max(logits, axis=0, keepdims=True)
    e = jnp.exp(logits - m)
    a = e / jnp.sum(e, axis=0, keepdims=True)

    xn_bf = xb * rnorm.astype(jnp.bfloat16)
    xfn = xn_bf.reshape(_C, _R, _C).reshape(_HW, _C)

    vlad = jax.lax.dot_general(
        a.astype(jnp.bfloat16), xfn, (((1,), (0,)), ((), ())),
        preferred_element_type=jnp.float32)
    vlad = vlad - jnp.sum(a, axis=1, keepdims=True) * cent_ref[...]
    n1 = jnp.sqrt(jnp.sum(vlad * vlad, axis=1, keepdims=True))
    vlad = vlad / jnp.maximum(n1, _EPS)
    n2 = jnp.sqrt(jnp.sum(vlad * vlad))
    out_ref[0] = vlad / jnp.maximum(n2, _EPS)


def kernel(x, conv_w, conv_b, centroids):
    x2 = x.reshape(_B, _C, _HW)
    out = pl.pallas_call(
        _netvlad_kernel,
        grid=(_B,),
        in_specs=[
            pl.BlockSpec((1, _C, _HW), lambda i: (i, 0, 0)),
            pl.BlockSpec((_K, _C), lambda i: (0, 0)),
            pl.BlockSpec((_K, 1), lambda i: (0, 0)),
            pl.BlockSpec((_K, _C), lambda i: (0, 0)),
        ],
        out_specs=pl.BlockSpec((1, _K, _C), lambda i: (i, 0, 0)),
        out_shape=jax.ShapeDtypeStruct((_B, _K, _C), jnp.float32),
        compiler_params=pltpu.CompilerParams(
            dimension_semantics=("parallel",),
            vmem_limit_bytes=56 * 1024 * 1024,
        ),
        name="netvlad_fused",
    )(x2, conv_w, conv_b.reshape(_K, 1), centroids)
    return out.reshape(_B, _K * _C)
